# Initial kernel scaffold; baseline (speedup 1.0000x reference)
import jax
import jax.numpy as jnp
from jax.experimental import pallas as pl


def kernel(x, edge_index, W_rel0, W_root0, b0, W_rel1, W_root1, b1, W_rel2, W_root2, b2):
    raise NotImplementedError("write your pallas kernel here")


# speedup vs baseline: 5.9344x; 5.9344x over previous
import functools

import jax
import jax.numpy as jnp
from jax import lax
from jax.experimental import pallas as pl
from jax.experimental.pallas import tpu as pltpu
from jax.experimental.pallas import tpu_sc as plsc

N = 10000
D = 128
N_CLS = 40
D2 = 128
N_PAD = 10240
N_TILES = 16
ROWS_PER_TILE = N_PAD // N_TILES
CHUNK = 128
E_ORIG = 320000
CHUNKS_PER_TILE = 79
E_PAD = 2 * N_TILES * CHUNKS_PER_TILE * CHUNK
EDGES_PER_CORE = E_PAD // 2
ROWS_BLK = 1000
GRID = N // ROWS_BLK


@functools.lru_cache(maxsize=None)
def _make_scatter(d):
  mesh = plsc.VectorSubcoreMesh(core_axis_name="c", subcore_axis_name="s",
                                num_cores=2, num_subcores=N_TILES)

  @functools.partial(
      pl.kernel,
      out_type=jax.ShapeDtypeStruct((2, N_PAD, d), jnp.float32),
      mesh=mesh,
      scratch_types=[
          pltpu.VMEM((CHUNK,), jnp.int32),
          pltpu.VMEM((CHUNK,), jnp.int32),
          pltpu.VMEM((CHUNK, d), jnp.float32),
          pltpu.VMEM_SHARED((N_PAD, d), jnp.float32),
          pltpu.SemaphoreType.DMA,
      ],
  )
  def scatter_kernel(y_hbm, src_hbm, dst_hbm, zeros_hbm, out_hbm,
                     src_v, dst_v, rows_v, acc_sh, sem):
    c = lax.axis_index("c")
    s = lax.axis_index("s")
    row0 = s * ROWS_PER_TILE
    pltpu.sync_copy(zeros_hbm, acc_sh.at[pl.ds(row0, ROWS_PER_TILE)])
    plsc.subcore_barrier()

    base = c * EDGES_PER_CORE + s * (CHUNKS_PER_TILE * CHUNK)

    def body(j, carry):
      e0 = base + j * CHUNK
      pltpu.sync_copy(src_hbm.at[pl.ds(e0, CHUNK)], src_v)
      pltpu.sync_copy(dst_hbm.at[pl.ds(e0, CHUNK)], dst_v)
      pltpu.async_copy(y_hbm.at[src_v], rows_v, sem).wait()
      pltpu.sync_copy(rows_v, acc_sh.at[dst_v], add=True)
      return carry

    lax.fori_loop(0, CHUNKS_PER_TILE, body, 0)
    plsc.subcore_barrier()
    pltpu.sync_copy(acc_sh.at[pl.ds(row0, ROWS_PER_TILE)],
                    out_hbm.at[c, pl.ds(row0, ROWS_PER_TILE)])

  return scatter_kernel


def _rows_spec(w):
  return pl.BlockSpec((ROWS_BLK, w), lambda i: (i, 0))


def _full_spec(r, w):
  return pl.BlockSpec((r, w), lambda i: (0, 0))


def _stage0_kernel(x_ref, wr_ref, wt_ref, b_ref, y_ref, r_ref):
  x = x_ref[...]
  y_ref[...] = jnp.dot(x, wr_ref[...], preferred_element_type=jnp.float32)
  r_ref[...] = jnp.dot(x, wt_ref[...], preferred_element_type=jnp.float32) + b_ref[...]


def _stage0(x, wr, wt, b):
  return pl.pallas_call(
      _stage0_kernel,
      grid=(GRID,),
      in_specs=[_rows_spec(D), _full_spec(D, D), _full_spec(D, D), _full_spec(1, D)],
      out_specs=[_rows_spec(D), _rows_spec(D)],
      out_shape=[jax.ShapeDtypeStruct((N, D), jnp.float32),
                 jax.ShapeDtypeStruct((N, D), jnp.float32)],
  )(x, wr, wt, b.reshape(1, D))


def _stage1_kernel(p0_ref, p1_ref, r0_ref, wr_ref, wt_ref, b_ref,
                   h_ref, y_ref, r_ref):
  h = jnp.maximum(p0_ref[...] + p1_ref[...] + r0_ref[...], 0.0)
  h_ref[...] = h
  y_ref[...] = jnp.dot(h, wr_ref[...], preferred_element_type=jnp.float32)
  r_ref[...] = jnp.dot(h, wt_ref[...], preferred_element_type=jnp.float32) + b_ref[...]


def _stage1(p0, p1, r0, wr, wt, b):
  return pl.pallas_call(
      _stage1_kernel,
      grid=(GRID,),
      in_specs=[_rows_spec(D), _rows_spec(D), _rows_spec(D),
                _full_spec(D, D), _full_spec(D, D), _full_spec(1, D)],
      out_specs=[_rows_spec(D), _rows_spec(D), _rows_spec(D)],
      out_shape=[jax.ShapeDtypeStruct((N, D), jnp.float32)] * 3,
  )(p0, p1, r0, wr, wt, b.reshape(1, D))


def _stage2_kernel(q0_ref, q1_ref, r1_ref, h0_ref, wr_ref, wt_ref, b_ref,
                   a_ref, y_ref, r_ref):
  h1 = jnp.maximum(q0_ref[...] + q1_ref[...] + r1_ref[...], 0.0)
  h0 = h0_ref[...]
  norm_prev = jnp.sqrt(jnp.sum(h0 * h0, axis=1, keepdims=True))
  norm_curr = jnp.sqrt(jnp.sum(h1 * h1, axis=1, keepdims=True))
  alpha = a_ref[...]
  scaled_prev = h0 * (norm_curr / (norm_prev + 1e-09))
  h = alpha * h1 + (1.0 - alpha) * scaled_prev
  y_ref[...] = jnp.dot(h, wr_ref[...], preferred_element_type=jnp.float32)
  r_ref[...] = jnp.dot(h, wt_ref[...], preferred_element_type=jnp.float32) + b_ref[...]


def _stage2(q0, q1, r1, h0, wr, wt, b, alpha_arr):
  return pl.pallas_call(
      _stage2_kernel,
      grid=(GRID,),
      in_specs=[_rows_spec(D), _rows_spec(D), _rows_spec(D), _rows_spec(D),
                _full_spec(D, D2), _full_spec(D, D2), _full_spec(1, D2),
                _full_spec(1, D)],
      out_specs=[_rows_spec(D2), _rows_spec(D2)],
      out_shape=[jax.ShapeDtypeStruct((N, D2), jnp.float32)] * 2,
  )(q0, q1, r1, h0, wr, wt, b, alpha_arr)


def _stage3_kernel(s0_ref, s1_ref, r2_ref, o_ref):
  o_ref[...] = s0_ref[...] + s1_ref[...] + r2_ref[...]


def _stage3(s0, s1, r2):
  return pl.pallas_call(
      _stage3_kernel,
      grid=(GRID,),
      in_specs=[_rows_spec(D2), _rows_spec(D2), _rows_spec(D2)],
      out_specs=_rows_spec(D2),
      out_shape=jax.ShapeDtypeStruct((N, D2), jnp.float32),
  )(s0, s1, r2)


def kernel(x, edge_index, W_rel0, W_root0, b0, W_rel1, W_root1, b1,
           W_rel2, W_root2, b2):
  src = edge_index[0]
  dst = edge_index[1]
  pad = E_PAD - E_ORIG
  pad_ar = jnp.arange(pad, dtype=jnp.int32)
  src_p = jnp.concatenate([src, pad_ar % N])
  dst_p = jnp.concatenate([dst, N + pad_ar % (N_PAD - N)])
  zeros128 = jnp.zeros((ROWS_PER_TILE, D), jnp.float32)
  zeros48 = jnp.zeros((ROWS_PER_TILE, D2), jnp.float32)
  alpha = jax.random.uniform(jax.random.key(42), (), dtype=jnp.float32)
  alpha_arr = jnp.full((1, D), alpha, jnp.float32)

  y0, root0 = _stage0(x, W_rel0, W_root0, b0)
  parts0 = _make_scatter(D)(y0, src_p, dst_p, zeros128)
  h0, y1, root1 = _stage1(parts0[0, :N], parts0[1, :N], root0,
                          W_rel1, W_root1, b1)
  parts1 = _make_scatter(D)(y1, src_p, dst_p, zeros128)
  wr2 = jnp.pad(W_rel2, ((0, 0), (0, D2 - N_CLS)))
  wt2 = jnp.pad(W_root2, ((0, 0), (0, D2 - N_CLS)))
  b2p = jnp.pad(b2, (0, D2 - N_CLS)).reshape(1, D2)
  y2, root2 = _stage2(parts1[0, :N], parts1[1, :N], root1, h0,
                      wr2, wt2, b2p, alpha_arr)
  parts2 = _make_scatter(D2)(y2, src_p, dst_p, zeros48)
  out = _stage3(parts2[0, :N], parts2[1, :N], root2)
  return out[:, :N_CLS]


# speedup vs baseline: 10.1215x; 1.7056x over previous
import functools

import jax
import jax.numpy as jnp
from jax import lax
from jax.experimental import pallas as pl
from jax.experimental.pallas import tpu as pltpu
from jax.experimental.pallas import tpu_sc as plsc

N = 10000
D = 128
N_CLS = 40
D2 = 128
N_PAD = 10240
N_TILES = 16
ROWS_PER_TILE = N_PAD // N_TILES
CHUNK = 128
E_ORIG = 320000
CHUNKS_PER_TILE = 80
E_PAD = 2 * N_TILES * CHUNKS_PER_TILE * CHUNK
EDGES_PER_CORE = E_PAD // 2
NBUF = 2
ROWS_BLK = 1000
GRID = N // ROWS_BLK


@functools.lru_cache(maxsize=None)
def _make_scatter(d):
  mesh = plsc.VectorSubcoreMesh(core_axis_name="c", subcore_axis_name="s",
                                num_cores=2, num_subcores=N_TILES)

  @functools.partial(
      pl.kernel,
      out_type=jax.ShapeDtypeStruct((2, N_PAD, d), jnp.float32),
      mesh=mesh,
      scratch_types=[
          pltpu.VMEM((CHUNK, d), jnp.float32),
          pltpu.VMEM((CHUNK, d), jnp.float32),
          pltpu.VMEM((CHUNK,), jnp.int32),
          pltpu.VMEM((CHUNK,), jnp.int32),
          pltpu.VMEM((CHUNK,), jnp.int32),
          pltpu.VMEM((CHUNK,), jnp.int32),
          pltpu.VMEM_SHARED((N_PAD, d), jnp.float32),
          pltpu.SemaphoreType.DMA,
          pltpu.SemaphoreType.DMA,
          pltpu.SemaphoreType.DMA,
          pltpu.SemaphoreType.DMA,
          pltpu.SemaphoreType.DMA,
          pltpu.SemaphoreType.DMA,
      ],
  )
  def scatter_kernel(y_hbm, src_hbm, dst_hbm, zeros_hbm, out_hbm,
                     rb0, rb1, si0, si1, di0, di1, acc_sh,
                     gs0, gs1, ss0, ss1, ds0, ds1):
    rows = (rb0, rb1)
    sidx = (si0, si1)
    didx = (di0, di1)
    gsem = (gs0, gs1)
    ssem = (ss0, ss1)
    dsem = (ds0, ds1)
    c = lax.axis_index("c")
    s = lax.axis_index("s")
    row0 = s * ROWS_PER_TILE
    base = (c * N_TILES + s) * (CHUNKS_PER_TILE * CHUNK)

    def idx_start(j, b):
      e0 = base + j * CHUNK
      pltpu.async_copy(src_hbm.at[pl.ds(e0, CHUNK)], sidx[b], ssem[b])
      pltpu.async_copy(dst_hbm.at[pl.ds(e0, CHUNK)], didx[b], dsem[b])

    def idx_wait(b):
      pltpu.make_async_copy(src_hbm.at[pl.ds(0, CHUNK)], sidx[b],
                            ssem[b]).wait()

    def gather_start(b):
      pltpu.async_copy(y_hbm.at[sidx[b]], rows[b], gsem[b])

    def gather_wait(b):
      pltpu.make_async_copy(y_hbm.at[sidx[b]], rows[b], gsem[b]).wait()

    def scatter(b):
      pltpu.make_async_copy(dst_hbm.at[pl.ds(0, CHUNK)], didx[b],
                            dsem[b]).wait()
      pltpu.sync_copy(rows[b], acc_sh.at[didx[b]], add=True)

    idx_start(0, 0)
    idx_start(1, 1)
    idx_wait(0)
    gather_start(0)
    pltpu.sync_copy(zeros_hbm, acc_sh.at[pl.ds(row0, ROWS_PER_TILE)])
    plsc.subcore_barrier()

    def step(j, b):
      gather_wait(b)
      idx_wait(1 - b)
      gather_start(1 - b)
      scatter(b)
      idx_start(j + 2, b)

    def body(i, carry):
      step(2 * i, 0)
      step(2 * i + 1, 1)
      return carry

    lax.fori_loop(0, (CHUNKS_PER_TILE - 2) // 2, body, 0)
    gather_wait(0)
    idx_wait(1)
    gather_start(1)
    scatter(0)
    gather_wait(1)
    scatter(1)
    plsc.subcore_barrier()
    pltpu.sync_copy(acc_sh.at[pl.ds(row0, ROWS_PER_TILE)],
                    out_hbm.at[c, pl.ds(row0, ROWS_PER_TILE)])

  return scatter_kernel


def _rows_spec(w):
  return pl.BlockSpec((ROWS_BLK, w), lambda i: (i, 0))


def _full_spec(r, w):
  return pl.BlockSpec((r, w), lambda i: (0, 0))


def _stage0_kernel(x_ref, wr_ref, wt_ref, b_ref, y_ref, r_ref):
  x = x_ref[...]
  y_ref[...] = jnp.dot(x, wr_ref[...], preferred_element_type=jnp.float32)
  r_ref[...] = jnp.dot(x, wt_ref[...], preferred_element_type=jnp.float32) + b_ref[...]


def _stage0(x, wr, wt, b):
  return pl.pallas_call(
      _stage0_kernel,
      grid=(GRID,),
      in_specs=[_rows_spec(D), _full_spec(D, D), _full_spec(D, D), _full_spec(1, D)],
      out_specs=[_rows_spec(D), _rows_spec(D)],
      out_shape=[jax.ShapeDtypeStruct((N, D), jnp.float32),
                 jax.ShapeDtypeStruct((N, D), jnp.float32)],
  )(x, wr, wt, b.reshape(1, D))


def _stage1_kernel(p0_ref, p1_ref, r0_ref, wr_ref, wt_ref, b_ref,
                   h_ref, y_ref, r_ref):
  h = jnp.maximum(p0_ref[...] + p1_ref[...] + r0_ref[...], 0.0)
  h_ref[...] = h
  y_ref[...] = jnp.dot(h, wr_ref[...], preferred_element_type=jnp.float32)
  r_ref[...] = jnp.dot(h, wt_ref[...], preferred_element_type=jnp.float32) + b_ref[...]


def _stage1(p0, p1, r0, wr, wt, b):
  return pl.pallas_call(
      _stage1_kernel,
      grid=(GRID,),
      in_specs=[_rows_spec(D), _rows_spec(D), _rows_spec(D),
                _full_spec(D, D), _full_spec(D, D), _full_spec(1, D)],
      out_specs=[_rows_spec(D), _rows_spec(D), _rows_spec(D)],
      out_shape=[jax.ShapeDtypeStruct((N, D), jnp.float32)] * 3,
  )(p0, p1, r0, wr, wt, b.reshape(1, D))


def _stage2_kernel(q0_ref, q1_ref, r1_ref, h0_ref, wr_ref, wt_ref, b_ref,
                   a_ref, y_ref, r_ref):
  h1 = jnp.maximum(q0_ref[...] + q1_ref[...] + r1_ref[...], 0.0)
  h0 = h0_ref[...]
  norm_prev = jnp.sqrt(jnp.sum(h0 * h0, axis=1, keepdims=True))
  norm_curr = jnp.sqrt(jnp.sum(h1 * h1, axis=1, keepdims=True))
  alpha = a_ref[...]
  scaled_prev = h0 * (norm_curr / (norm_prev + 1e-09))
  h = alpha * h1 + (1.0 - alpha) * scaled_prev
  y_ref[...] = jnp.dot(h, wr_ref[...], preferred_element_type=jnp.float32)
  r_ref[...] = jnp.dot(h, wt_ref[...], preferred_element_type=jnp.float32) + b_ref[...]


def _stage2(q0, q1, r1, h0, wr, wt, b, alpha_arr):
  return pl.pallas_call(
      _stage2_kernel,
      grid=(GRID,),
      in_specs=[_rows_spec(D), _rows_spec(D), _rows_spec(D), _rows_spec(D),
                _full_spec(D, D2), _full_spec(D, D2), _full_spec(1, D2),
                _full_spec(1, D)],
      out_specs=[_rows_spec(D2), _rows_spec(D2)],
      out_shape=[jax.ShapeDtypeStruct((N, D2), jnp.float32)] * 2,
  )(q0, q1, r1, h0, wr, wt, b, alpha_arr)


def _stage3_kernel(s0_ref, s1_ref, r2_ref, o_ref):
  o_ref[...] = s0_ref[...] + s1_ref[...] + r2_ref[...]


def _stage3(s0, s1, r2):
  return pl.pallas_call(
      _stage3_kernel,
      grid=(GRID,),
      in_specs=[_rows_spec(D2), _rows_spec(D2), _rows_spec(D2)],
      out_specs=_rows_spec(D2),
      out_shape=jax.ShapeDtypeStruct((N, D2), jnp.float32),
  )(s0, s1, r2)


def kernel(x, edge_index, W_rel0, W_root0, b0, W_rel1, W_root1, b1,
           W_rel2, W_root2, b2):
  src = edge_index[0]
  dst = edge_index[1]
  pad = E_PAD - E_ORIG
  pad_ar = jnp.arange(pad, dtype=jnp.int32)
  src_p = jnp.concatenate([src, pad_ar % N])
  dst_p = jnp.concatenate([dst, N + pad_ar % (N_PAD - N)])
  zeros128 = jnp.zeros((ROWS_PER_TILE, D), jnp.float32)
  zeros48 = jnp.zeros((ROWS_PER_TILE, D2), jnp.float32)
  alpha = jax.random.uniform(jax.random.key(42), (), dtype=jnp.float32)
  alpha_arr = jnp.full((1, D), alpha, jnp.float32)

  y0, root0 = _stage0(x, W_rel0, W_root0, b0)
  parts0 = _make_scatter(D)(y0, src_p, dst_p, zeros128)
  h0, y1, root1 = _stage1(parts0[0, :N], parts0[1, :N], root0,
                          W_rel1, W_root1, b1)
  parts1 = _make_scatter(D)(y1, src_p, dst_p, zeros128)
  wr2 = jnp.pad(W_rel2, ((0, 0), (0, D2 - N_CLS)))
  wt2 = jnp.pad(W_root2, ((0, 0), (0, D2 - N_CLS)))
  b2p = jnp.pad(b2, (0, D2 - N_CLS)).reshape(1, D2)
  y2, root2 = _stage2(parts1[0, :N], parts1[1, :N], root1, h0,
                      wr2, wt2, b2p, alpha_arr)
  parts2 = _make_scatter(D2)(y2, src_p, dst_p, zeros48)
  out = _stage3(parts2[0, :N], parts2[1, :N], root2)
  return out[:, :N_CLS]
